# baseline (speedup 1.0000x reference)
import functools

import jax
import jax.numpy as jnp
from jax import lax
from jax.experimental import pallas as pl
from jax.experimental.pallas import tpu as pltpu
from jax.experimental.pallas import tpu_sc as plsc

_N_USERS = 1000000
_N_ITEMS = 100000
_N_GENRES = 26
_K = 16
_B = 16384


def _sc_gather(p_rows, q_rows, user_ids, item_ids):
    mesh = plsc.VectorSubcoreMesh(core_axis_name="c", subcore_axis_name="s")

    @functools.partial(
        pl.kernel,
        mesh=mesh,
        out_type=(
            jax.ShapeDtypeStruct((_K, _B), jnp.float32),
            jax.ShapeDtypeStruct((_K, _B), jnp.float32),
        ),
        scratch_types=[
            pltpu.VMEM((_B,), jnp.int32),
            pltpu.VMEM((_B,), jnp.float32),
            pltpu.SemaphoreType.DMA,
        ],
    )
    def gather_kernel(*refs):
        p_refs = refs[:_K]
        q_refs = refs[_K:2 * _K]
        ui_hbm, ii_hbm, pu_hbm, qi_hbm, idx_v, row_v, sem = refs[2 * _K:]
        wid = lax.axis_index("s") * 2 + lax.axis_index("c")

        for k in range(_K):
            @pl.when(wid == k)
            def _():
                pltpu.sync_copy(ui_hbm, idx_v)
                pltpu.async_copy(p_refs[k].at[idx_v], row_v, sem).wait()
                pltpu.sync_copy(row_v, pu_hbm.at[k])

            @pl.when(wid == _K + k)
            def _():
                pltpu.sync_copy(ii_hbm, idx_v)
                pltpu.async_copy(q_refs[k].at[idx_v], row_v, sem).wait()
                pltpu.sync_copy(row_v, qi_hbm.at[k])

    return gather_kernel(*p_rows, *q_rows, user_ids, item_ids)


def _combine_body(put_ref, qit_ref, g_ref, wg_ref, o_ref):
    qgt = lax.dot_general(
        wg_ref[...], g_ref[...],
        dimension_numbers=(((1,), (1,)), ((), ())),
        preferred_element_type=jnp.float32,
    )
    o_ref[...] = jnp.sum(put_ref[...] * (qit_ref[...] + qgt), axis=0)


def _tc_combine(put, qit, genres, wg):
    blk = 2048
    grid = (_B // blk,)
    return pl.pallas_call(
        _combine_body,
        out_shape=jax.ShapeDtypeStruct((_B,), jnp.float32),
        grid=grid,
        in_specs=[
            pl.BlockSpec((_K, blk), lambda i: (0, i)),
            pl.BlockSpec((_K, blk), lambda i: (0, i)),
            pl.BlockSpec((blk, _N_GENRES), lambda i: (i, 0)),
            pl.BlockSpec((_K, _N_GENRES), lambda i: (0, 0)),
        ],
        out_specs=pl.BlockSpec((blk,), lambda i: (i,)),
    )(put, qit, genres, wg)


def kernel(user_ids, item_ids, genres_one_hot, P_w, Q_w):
    p_rows = [P_w[k] for k in range(_K)]
    q_rows = [Q_w[k] for k in range(_K)]
    wg = Q_w[:, _N_ITEMS:]

    put, qit = _sc_gather(p_rows, q_rows, user_ids.astype(jnp.int32),
                          item_ids.astype(jnp.int32))
    return _tc_combine(put, qit, genres_one_hot, wg).reshape(_B, 1)


# speedup vs baseline: 1.5310x; 1.5310x over previous
import functools

import jax
import jax.numpy as jnp
from jax import lax
from jax.experimental import pallas as pl
from jax.experimental.pallas import tpu as pltpu
from jax.experimental.pallas import tpu_sc as plsc

_N_USERS = 1000000
_N_ITEMS = 100000
_N_GENRES = 26
_K = 16
_B = 16384

_NPIECES = 8
_KPP = _K // _NPIECES
_VEC = 16


def _sc_gather(p_pieces, q_flat, user_ids, item_ids):
    mesh = plsc.VectorSubcoreMesh(core_axis_name="c", subcore_axis_name="s")

    @functools.partial(
        pl.kernel,
        mesh=mesh,
        out_type=(
            jax.ShapeDtypeStruct((_K, _B), jnp.float32),
            jax.ShapeDtypeStruct((_K, _B), jnp.float32),
        ),
        scratch_types=[
            pltpu.VMEM((_B,), jnp.int32),
            pltpu.VMEM((_B,), jnp.int32),
            pltpu.VMEM((_B,), jnp.float32),
            pltpu.SemaphoreType.DMA,
        ],
    )
    def gather_kernel(*refs):
        p_refs = refs[:_NPIECES]
        (q_hbm, ui_hbm, ii_hbm, pu_hbm, qi_hbm,
         ids_v, idx_v, row_v, sem) = refs[_NPIECES:]
        wid = lax.axis_index("s") * 2 + lax.axis_index("c")
        k = lax.rem(wid, _K)

        def add_offset(offset):
            off_vec = jax.lax.broadcast(offset, (_VEC,))

            @pl.loop(0, _B, step=_VEC)
            def _(j):
                idx_v[pl.ds(j, _VEC)] = ids_v[pl.ds(j, _VEC)] + off_vec

        @pl.when(wid < _K)
        def _():
            pltpu.sync_copy(ui_hbm, ids_v)
            add_offset(lax.rem(k, _KPP) * _N_USERS)
            for pc in range(_NPIECES):
                @pl.when(k // _KPP == pc)
                def _():
                    pltpu.async_copy(p_refs[pc].at[idx_v], row_v, sem).wait()
            pltpu.sync_copy(row_v, pu_hbm.at[k])

        @pl.when(wid >= _K)
        def _():
            pltpu.sync_copy(ii_hbm, ids_v)
            add_offset(k * (_N_ITEMS + _N_GENRES))
            pltpu.async_copy(q_hbm.at[idx_v], row_v, sem).wait()
            pltpu.sync_copy(row_v, qi_hbm.at[k])

    return gather_kernel(*p_pieces, q_flat, user_ids, item_ids)


def _combine_body(put_ref, qit_ref, g_ref, wg_ref, o_ref):
    qgt = lax.dot_general(
        wg_ref[...], g_ref[...],
        dimension_numbers=(((1,), (1,)), ((), ())),
        preferred_element_type=jnp.float32,
    )
    o_ref[...] = jnp.sum(put_ref[...] * (qit_ref[...] + qgt), axis=0)


def _tc_combine(put, qit, genres, wg):
    blk = 2048
    grid = (_B // blk,)
    return pl.pallas_call(
        _combine_body,
        out_shape=jax.ShapeDtypeStruct((_B,), jnp.float32),
        grid=grid,
        in_specs=[
            pl.BlockSpec((_K, blk), lambda i: (0, i)),
            pl.BlockSpec((_K, blk), lambda i: (0, i)),
            pl.BlockSpec((blk, _N_GENRES), lambda i: (i, 0)),
            pl.BlockSpec((_K, _N_GENRES), lambda i: (0, 0)),
        ],
        out_specs=pl.BlockSpec((blk,), lambda i: (i,)),
    )(put, qit, genres, wg)


def kernel(user_ids, item_ids, genres_one_hot, P_w, Q_w):
    p_pieces = [P_w[pc * _KPP:(pc + 1) * _KPP].reshape(-1)
                for pc in range(_NPIECES)]
    q_flat = Q_w.reshape(-1)
    wg = Q_w[:, _N_ITEMS:]

    put, qit = _sc_gather(p_pieces, q_flat, user_ids.astype(jnp.int32),
                          item_ids.astype(jnp.int32))
    return _tc_combine(put, qit, genres_one_hot, wg).reshape(_B, 1)


# speedup vs baseline: 1.5438x; 1.0083x over previous
import functools

import jax
import jax.numpy as jnp
from jax import lax
from jax.experimental import pallas as pl
from jax.experimental.pallas import tpu as pltpu
from jax.experimental.pallas import tpu_sc as plsc

_N_USERS = 1000000
_N_ITEMS = 100000
_N_GENRES = 26
_K = 16
_B = 16384

_NPIECES = 8
_KPP = _K // _NPIECES
_VEC = 16


_HB = _B // 2


def _sc_gather_p(p_pieces, user_ids):
    mesh = plsc.VectorSubcoreMesh(core_axis_name="c", subcore_axis_name="s")

    @functools.partial(
        pl.kernel,
        mesh=mesh,
        out_type=jax.ShapeDtypeStruct((_K, _B), jnp.float32),
        scratch_types=[
            pltpu.VMEM((_HB,), jnp.int32),
            pltpu.VMEM((_HB,), jnp.int32),
            pltpu.VMEM((_HB,), jnp.float32),
            pltpu.SemaphoreType.DMA,
        ],
    )
    def gather_kernel(*refs):
        p_refs = refs[:_NPIECES]
        ui_hbm, pu_hbm, ids_v, idx_v, row_v, sem = refs[_NPIECES:]
        wid = lax.axis_index("s") * 2 + lax.axis_index("c")
        k = wid // 2
        half = lax.rem(wid, 2)
        base = half * _HB
        off_vec = jax.lax.broadcast(lax.rem(k, _KPP) * _N_USERS, (_VEC,))

        pltpu.sync_copy(ui_hbm.at[pl.ds(base, _HB)], ids_v)

        @pl.loop(0, _HB, step=_VEC)
        def _(j):
            idx_v[pl.ds(j, _VEC)] = ids_v[pl.ds(j, _VEC)] + off_vec

        for pc in range(_NPIECES):
            @pl.when(k // _KPP == pc)
            def _():
                pltpu.async_copy(p_refs[pc].at[idx_v], row_v, sem).wait()
        pltpu.sync_copy(row_v, pu_hbm.at[k, pl.ds(base, _HB)])

    return gather_kernel(*p_pieces, user_ids)


def _sc_gather_q(q_flat, item_ids):
    mesh = plsc.VectorSubcoreMesh(core_axis_name="c", subcore_axis_name="s")

    @functools.partial(
        pl.kernel,
        mesh=mesh,
        out_type=jax.ShapeDtypeStruct((_K, _B), jnp.float32),
        scratch_types=[
            pltpu.VMEM((_HB,), jnp.int32),
            pltpu.VMEM((_HB,), jnp.int32),
            pltpu.VMEM((_HB,), jnp.float32),
            pltpu.SemaphoreType.DMA,
        ],
    )
    def gather_kernel(q_hbm, ii_hbm, qi_hbm, ids_v, idx_v, row_v, sem):
        wid = lax.axis_index("s") * 2 + lax.axis_index("c")
        k = wid // 2
        half = lax.rem(wid, 2)
        base = half * _HB
        off_vec = jax.lax.broadcast(k * (_N_ITEMS + _N_GENRES), (_VEC,))

        pltpu.sync_copy(ii_hbm.at[pl.ds(base, _HB)], ids_v)

        @pl.loop(0, _HB, step=_VEC)
        def _(j):
            idx_v[pl.ds(j, _VEC)] = ids_v[pl.ds(j, _VEC)] + off_vec

        pltpu.async_copy(q_hbm.at[idx_v], row_v, sem).wait()
        pltpu.sync_copy(row_v, qi_hbm.at[k, pl.ds(base, _HB)])

    return gather_kernel(q_flat, item_ids)


def _combine_body(put_ref, qit_ref, g_ref, wg_ref, o_ref):
    qgt = lax.dot_general(
        wg_ref[...], g_ref[...],
        dimension_numbers=(((1,), (1,)), ((), ())),
        preferred_element_type=jnp.float32,
    )
    o_ref[...] = jnp.sum(put_ref[...] * (qit_ref[...] + qgt), axis=0)


def _tc_combine(put, qit, genres, wg):
    blk = 2048
    grid = (_B // blk,)
    return pl.pallas_call(
        _combine_body,
        out_shape=jax.ShapeDtypeStruct((_B,), jnp.float32),
        grid=grid,
        in_specs=[
            pl.BlockSpec((_K, blk), lambda i: (0, i)),
            pl.BlockSpec((_K, blk), lambda i: (0, i)),
            pl.BlockSpec((blk, _N_GENRES), lambda i: (i, 0)),
            pl.BlockSpec((_K, _N_GENRES), lambda i: (0, 0)),
        ],
        out_specs=pl.BlockSpec((blk,), lambda i: (i,)),
    )(put, qit, genres, wg)


def kernel(user_ids, item_ids, genres_one_hot, P_w, Q_w):
    p_pieces = [P_w[pc * _KPP:(pc + 1) * _KPP].reshape(-1)
                for pc in range(_NPIECES)]
    q_flat = Q_w.reshape(-1)
    wg = Q_w[:, _N_ITEMS:]

    qit = _sc_gather_q(q_flat, item_ids.astype(jnp.int32))
    put = _sc_gather_p(p_pieces, user_ids.astype(jnp.int32))
    return _tc_combine(put, qit, genres_one_hot, wg).reshape(_B, 1)
